# baseline (speedup 1.0000x reference)
import functools

import jax
import jax.numpy as jnp
from jax import lax
from jax.experimental import pallas as pl
from jax.experimental.pallas import tpu as pltpu
from jax.experimental.pallas import tpu_sc as plsc

EMB = 64
HID = 128
TAGS = 50

NC = 2
NS = 16
NW = NC * NS
CHUNK = 128


def _sc_gather(emb, idx3):
    nchunk = idx3.shape[1]
    mesh = plsc.VectorSubcoreMesh(core_axis_name="c", subcore_axis_name="s")

    @functools.partial(
        pl.kernel,
        out_type=jax.ShapeDtypeStruct((NW, nchunk, CHUNK, EMB), jnp.float32),
        mesh=mesh,
        scratch_types=[
            pltpu.VMEM((nchunk, CHUNK), jnp.int32),
            pltpu.VMEM((nchunk, CHUNK, EMB), jnp.float32),
            pltpu.SemaphoreType.DMA,
        ],
        compiler_params=pltpu.CompilerParams(use_tc_tiling_on_sc=False),
    )
    def k(table_hbm, idx_hbm, out_hbm, idx_v, rows_v, sem):
        wid = lax.axis_index("s") * NC + lax.axis_index("c")
        pltpu.sync_copy(idx_hbm.at[wid], idx_v)
        copies = [
            pltpu.async_copy(table_hbm.at[idx_v.at[j]], rows_v.at[j], sem)
            for j in range(nchunk)
        ]
        for c in copies:
            c.wait()
        pltpu.sync_copy(rows_v, out_hbm.at[wid])

    return k(emb, idx3)


def _mlp_body(e_ref, w1_ref, b1_ref, w2_ref, b2_ref, o_ref):
    h = jnp.dot(e_ref[...], w1_ref[...], preferred_element_type=jnp.float32)
    h = h + b1_ref[...]
    t = jnp.dot(h, w2_ref[...], preferred_element_type=jnp.float32)
    t = t + b2_ref[...]
    x = t - jnp.max(t, axis=1, keepdims=True)
    o_ref[...] = x - jnp.log(jnp.sum(jnp.exp(x), axis=1, keepdims=True))


def _mlp(embeds, W1, b1, W2, b2, interpret=False):
    n = embeds.shape[0]
    blk = min(n, 2048)
    return pl.pallas_call(
        _mlp_body,
        grid=(n // blk,),
        in_specs=[
            pl.BlockSpec((blk, EMB), lambda i: (i, 0)),
            pl.BlockSpec((EMB, HID), lambda i: (0, 0)),
            pl.BlockSpec((1, HID), lambda i: (0, 0)),
            pl.BlockSpec((HID, TAGS), lambda i: (0, 0)),
            pl.BlockSpec((1, TAGS), lambda i: (0, 0)),
        ],
        out_specs=pl.BlockSpec((blk, TAGS), lambda i: (i, 0)),
        out_shape=jax.ShapeDtypeStruct((n, TAGS), jnp.float32),
        interpret=interpret,
    )(embeds, W1, b1.reshape(1, HID), W2, b2.reshape(1, TAGS))


def kernel(sentence, emb, W1, b1, W2, b2):
    n = sentence.shape[0]
    nchunk = n // (NW * CHUNK)
    idx3 = sentence.astype(jnp.int32).reshape(NW, nchunk, CHUNK)
    rows = _sc_gather(emb, idx3).reshape(n, EMB)
    return _mlp(rows, W1, b1, W2, b2)
